# baseline (speedup 1.0000x reference)
import functools

import numpy as np

import jax
import jax.numpy as jnp
from jax import lax
from jax.experimental import pallas as pl
from jax.experimental.pallas import tpu as pltpu
from jax.experimental.pallas import tpu_sc as plsc


def _prep_body(aa_ref, pos_ref, pc_ref, pcW_ref, pcb_ref, sv_ref,
               ev_ref, sW1_ref, sb1_ref, sW2_ref, sb2_ref, eW1_ref, eb1_ref,
               eW2_ref, eb2_ref, nW_ref, nb_ref, perm_ref, TP_out, S_out):
    hp = jax.lax.Precision.HIGHEST
    L = TP_out.shape[0] // aa_ref.shape[0]
    nW = nW_ref[...]
    nW_aa, nW_pos, nW_pc = nW[0:32, :], nW[32:48, :], nW[48:64, :]
    nW_st, nW_ev = nW[64:96, :], nW[96:128, :]

    pc_feat = jnp.dot(pc_ref[...], pcW_ref[...], precision=hp) + pcb_ref[...]
    T = (jnp.dot(aa_ref[...], nW_aa, precision=hp)
         + jnp.dot(pc_feat, nW_pc, precision=hp))
    P = jnp.dot(pos_ref[0:L, :], nW_pos, precision=hp)
    V, H = T.shape
    TP = (P[:, None, :] + T[None, :, :]).reshape(L * V, H)
    TP_out[...] = jnp.dot(TP, perm_ref[...],
                          precision=hp).astype(jnp.bfloat16)

    sv = sv_ref[...]
    f = jnp.concatenate([
        sv[:, 0:1] * 0.1,
        sv[:, 1:2] * (1.0 / 2000.0),
        jnp.log1p(jnp.maximum(sv[:, 2:3], 0.0)) * (1.0 / 20.0),
    ], axis=1)
    f = jnp.nan_to_num(f, nan=0.0, posinf=10.0, neginf=-10.0)
    hs = jnp.maximum(jnp.dot(f, sW1_ref[...], precision=hp) + sb1_ref[...], 0.0)
    s32 = jnp.dot(hs, sW2_ref[...], precision=hp) + sb2_ref[...]

    e = ev_ref[...] * 0.01
    e = jnp.nan_to_num(e, nan=0.0, posinf=10.0, neginf=-10.0)
    he = jnp.maximum(jnp.dot(e, eW1_ref[...], precision=hp) + eb1_ref[...], 0.0)
    e32 = jnp.dot(he, eW2_ref[...], precision=hp) + eb2_ref[...]

    S_out[...] = (jnp.dot(s32, nW_st, precision=hp)
                  + jnp.dot(e32, nW_ev, precision=hp) + nb_ref[...])


def _make_sc_main(B, L, H, NC, NS):
    NW = NC * NS
    BPW = B // NW
    NJ = H // 16
    LP = L // 2
    f32 = jnp.float32

    NR = 8

    @functools.partial(
        pl.kernel,
        out_type=jax.ShapeDtypeStruct((B, L, H), f32),
        mesh=plsc.VectorSubcoreMesh(core_axis_name="c", subcore_axis_name="s"),
        scratch_types=[
            pltpu.VMEM((BPW, LP), jnp.int32),
            pltpu.VMEM((BPW, H), f32),
            pltpu.VMEM((NR, LP, H), f32),
            pltpu.VMEM((NR, L, H), f32),
            pltpu.VMEM((1, H), f32),
            pltpu.VMEM((1, H), f32),
        ] + [pltpu.SemaphoreType.DMA] * (2 * NR),
    )
    def sc_main(TP_hbm, idx_hbm, S_hbm, gam_hbm, bet_hbm, out_hbm,
                idx_v, S_v, gb, ob, gam_v, bet_v, *sems):
        sgs = sems[:NR]
        sos = sems[NR:]
        wid = lax.axis_index("s") * NC + lax.axis_index("c")
        b0 = wid * BPW
        pltpu.sync_copy(idx_hbm.at[pl.ds(b0, BPW)], idx_v)
        pltpu.sync_copy(S_hbm.at[pl.ds(b0, BPW)], S_v)
        pltpu.sync_copy(gam_hbm, gam_v)
        pltpu.sync_copy(bet_hbm, bet_v)

        gam = [gam_v[0, pl.ds(16 * j, 16)] for j in range(NJ)]
        bet = [bet_v[0, pl.ds(16 * j, 16)] for j in range(NJ)]
        lanes = lax.iota(jnp.int32, 16)
        perms = [(lanes ^ c)[:, None] for c in (8, 4, 2, 1)]
        dnums = lax.GatherDimensionNumbers(
            offset_dims=(), collapsed_slice_dims=(0,), start_index_map=(0,))

        def lane_swap(v, perm):
            return lax.gather(v, perm, dnums, slice_sizes=(1,),
                              mode=lax.GatherScatterMode.PROMISE_IN_BOUNDS)

        for r in range(NR - 1):
            pltpu.async_copy(TP_hbm.at[idx_v.at[r]], gb.at[r], sgs[r])

        def do_b(b, p):
            pltpu.make_async_copy(TP_hbm.at[idx_v.at[b]], gb.at[p],
                                  sgs[p]).wait()

            pn = (p + NR - 1) % NR

            @pl.when(b + NR - 1 < BPW)
            def _():
                pltpu.async_copy(TP_hbm.at[idx_v.at[b + NR - 1]], gb.at[pn],
                                 sgs[pn])

            @pl.when(b >= NR)
            def _():
                pltpu.make_async_copy(ob.at[p], out_hbm.at[b0 + b - NR],
                                      sos[p]).wait()

            Sb = [S_v[b, pl.ds(16 * j, 16)] for j in range(NJ)]
            gbp = gb.at[p]
            obp = ob.at[p]

            @plsc.parallel_loop(0, LP, unroll=1)
            def rowpair(m):
                for half in range(2):
                    x = []
                    for jj in range(NJ // 2):
                        wi = lax.bitcast_convert_type(
                            gbp[m, pl.ds(64 * half + 16 * jj, 16)], jnp.int32)
                        lo = lax.bitcast_convert_type(wi << 16, f32)
                        hi = lax.bitcast_convert_type(
                            wi & jnp.int32(-65536), f32)
                        x.append(lo + Sb[2 * jj])
                        x.append(hi + Sb[2 * jj + 1])
                    s = (((x[0] + x[1]) + (x[2] + x[3]))
                         + ((x[4] + x[5]) + (x[6] + x[7])))
                    q = ((((x[0] * x[0] + x[1] * x[1])
                           + (x[2] * x[2] + x[3] * x[3]))
                          + ((x[4] * x[4] + x[5] * x[5])
                             + (x[6] * x[6] + x[7] * x[7]))))
                    for perm in perms:
                        s = s + lane_swap(s, perm)
                        q = q + lane_swap(q, perm)
                    mu = s * (1.0 / H)
                    var = q * (1.0 / H) - mu * mu
                    a = var + 1e-5
                    ai = lax.bitcast_convert_type(a, jnp.int32)
                    y = lax.bitcast_convert_type(
                        jnp.int32(0x5F375A86) - (ai >> 1), f32)
                    y = y * (1.5 - 0.5 * a * y * y)
                    for j in range(NJ):
                        obp[2 * m + half, pl.ds(16 * j, 16)] = jnp.maximum(
                            (x[j] - mu) * y * gam[j] + bet[j], 0.0)
            pltpu.async_copy(obp, out_hbm.at[b0 + b], sos[p])

        def bodyn(i, carry):
            for r in range(NR):
                do_b(NR * i + r, r)
            return carry

        lax.fori_loop(0, BPW // NR, bodyn, 0)
        for r in range(NR):
            pltpu.make_async_copy(ob.at[r], out_hbm.at[b0 + BPW - NR + r],
                                  sos[r]).wait()

    return sc_main


def kernel(seq_tokens, state_vars, env_vars, aa_emb, pos_emb, pc_table, pc_W,
           pc_b, sW1, sb1, sW2, sb2, eW1, eb1, eW2, eb2, nW, nb, gamma, beta):
    B, L = seq_tokens.shape
    V, H = aa_emb.shape[0], nW.shape[1]
    f32 = jnp.float32

    sigma = np.zeros(H, np.int64)
    for jj in range(H // 32):
        for k in range(16):
            sigma[32 * jj + k] = 32 * jj + 2 * k
            sigma[32 * jj + 16 + k] = 32 * jj + 2 * k + 1
    pmat = np.zeros((H, H), np.float32)
    pmat[np.arange(H), sigma] = 1.0

    TP, S = pl.pallas_call(
        _prep_body,
        out_shape=[
            jax.ShapeDtypeStruct((L * V, H), jnp.bfloat16),
            jax.ShapeDtypeStruct((B, H), f32),
        ],
    )(aa_emb, pos_emb, pc_table, pc_W, pc_b.reshape(1, -1),
      state_vars, env_vars, sW1, sb1.reshape(1, -1), sW2, sb2.reshape(1, -1),
      eW1, eb1.reshape(1, -1), eW2, eb2.reshape(1, -1), nW, nb.reshape(1, -1),
      jnp.asarray(pmat))

    TP2 = lax.bitcast_convert_type(TP.reshape(L * V, H // 2, 2), f32)
    TPr = TP2.reshape(L // 2, 2, V, H // 2)
    pair = jnp.concatenate([
        jnp.broadcast_to(TPr[:, 0][:, :, None, :], (L // 2, V, V, H // 2)),
        jnp.broadcast_to(TPr[:, 1][:, None, :, :], (L // 2, V, V, H // 2)),
    ], axis=-1).reshape(L // 2 * V * V, H)
    tok = seq_tokens.astype(jnp.int32)
    idxp = (jnp.arange(L // 2, dtype=jnp.int32) * (V * V))[None, :] \
        + tok[:, 0::2] * V + tok[:, 1::2]

    info = plsc.get_sparse_core_info()
    sc_main = _make_sc_main(B, L, H, info.num_cores, info.num_subcores)
    return sc_main(pair, idxp, S, gamma.reshape(1, -1), beta.reshape(1, -1))
